# baseline (speedup 1.0000x reference)
import functools

import jax
import jax.numpy as jnp
from jax import lax
from jax.experimental import pallas as pl
from jax.experimental.pallas import tpu as pltpu
from jax.experimental.pallas import tpu_sc as plsc

_EPS = 1e-5
_NC = 2
_NS = 16
_NW = _NC * _NS
_K = 128
_CPW = 80
_NBUF = 2


def _deg_sc(dst2d, n, npad):
    mesh = plsc.VectorSubcoreMesh(core_axis_name="c", subcore_axis_name="s")

    @functools.partial(
        pl.kernel,
        out_type=jax.ShapeDtypeStruct((_NC * npad,), jnp.float32),
        mesh=mesh,
        scratch_types=[
            pltpu.VMEM((_CPW, _K), jnp.int32),
            pltpu.VMEM((_K,), jnp.float32),
            pltpu.VMEM((npad,), jnp.float32),
            pltpu.VMEM_SHARED((npad,), jnp.float32),
            pltpu.SemaphoreType.DMA,
        ],
    )
    def body(dst_hbm, out_hbm, didx, ones, zbuf, acc, sem):
        cid = lax.axis_index("c")
        sid = lax.axis_index("s")
        wid = sid * _NC + cid
        pltpu.sync_copy(dst_hbm.at[pl.ds(wid * _CPW, _CPW)], didx)
        for j in range(_K // 16):
            ones[pl.ds(j * 16, 16)] = jnp.full((16,), 1.0, jnp.float32)

        @pl.when(sid == 0)
        def _zero():
            def zstep(i, c):
                zbuf[pl.ds(i * 16, 16)] = jnp.zeros((16,), jnp.float32)
                return c
            lax.fori_loop(0, npad // 16, zstep, 0)
            pltpu.sync_copy(zbuf, acc)

        plsc.subcore_barrier()

        def step(i, c):
            pltpu.async_copy(ones, acc.at[didx.at[i]], sem, add=True)
            return c

        lax.fori_loop(0, _CPW, step, 0)

        def drain(i, c):
            pltpu.make_async_copy(ones, acc.at[didx.at[0]], sem).wait()
            return c

        lax.fori_loop(0, _CPW, drain, 0)
        plsc.subcore_barrier()

        @pl.when(sid == 0)
        def _out():
            pltpu.sync_copy(acc, out_hbm.at[pl.ds(cid * npad, npad)])

    return body(dst2d)


def _seg_sum_sc(g, src2d, dst2d, zeros):
    n, d = g.shape
    nacc = zeros.shape[0]
    rpt = (n // _NS) // 8 * 8
    ztail = nacc - _NS * rpt
    otail = n - _NS * rpt
    cpq = 16
    nq = _CPW // cpq
    mesh = plsc.VectorSubcoreMesh(core_axis_name="c", subcore_axis_name="s")

    @functools.partial(
        pl.kernel,
        out_type=jax.ShapeDtypeStruct((_NC, n, d), jnp.float32),
        mesh=mesh,
        scratch_types=(
            [pltpu.VMEM((cpq, _K), jnp.int32),
             pltpu.VMEM((cpq, _K), jnp.int32),
             pltpu.VMEM_SHARED((nacc, d), jnp.float32)]
            + [pltpu.VMEM((_K, d), jnp.float32) for _ in range(_NBUF)]
            + [pltpu.SemaphoreType.DMA for _ in range(2 * _NBUF)]
        ),
    )
    def body(g_hbm, src_hbm, dst_hbm, z_hbm, out_hbm, sidx, didx, acc, *bufs):
        rows = bufs[:_NBUF]
        gsem = bufs[_NBUF:2 * _NBUF]
        ssem = bufs[2 * _NBUF:]
        cid = lax.axis_index("c")
        sid = lax.axis_index("s")
        wid = sid * _NC + cid

        pltpu.sync_copy(z_hbm.at[pl.ds(sid * rpt, rpt)],
                        acc.at[pl.ds(sid * rpt, rpt)])

        @pl.when(sid == 0)
        def _ztail():
            pltpu.sync_copy(z_hbm.at[pl.ds(_NS * rpt, ztail)],
                            acc.at[pl.ds(_NS * rpt, ztail)])

        plsc.subcore_barrier()

        nblk = cpq // _NBUF
        for q in range(nq):
            qbase = wid * _CPW + q * cpq
            pltpu.sync_copy(src_hbm.at[pl.ds(qbase, cpq)], sidx)
            pltpu.sync_copy(dst_hbm.at[pl.ds(qbase, cpq)], didx)
            for b in range(_NBUF):
                pltpu.async_copy(g_hbm.at[sidx.at[b]], rows[b], gsem[b])

            def blk(k, c):
                j0 = k * _NBUF
                for b in range(_NBUF):
                    j = j0 + b
                    pltpu.make_async_copy(g_hbm.at[sidx.at[j]], rows[b],
                                          gsem[b]).wait()
                    pltpu.async_copy(rows[b], acc.at[didx.at[j]], ssem[b],
                                     add=True)
                for b in range(_NBUF):
                    j = j0 + b

                    @pl.when(k < nblk - 1)
                    def _next():
                        pltpu.make_async_copy(rows[b], acc.at[didx.at[j]],
                                              ssem[b]).wait()
                        pltpu.async_copy(g_hbm.at[sidx.at[j + _NBUF]],
                                         rows[b], gsem[b])
                return c

            lax.fori_loop(0, nblk, blk, 0)
            for b in range(_NBUF):
                pltpu.make_async_copy(rows[b], acc.at[didx.at[0]],
                                      ssem[b]).wait()
        plsc.subcore_barrier()

        pltpu.sync_copy(acc.at[pl.ds(sid * rpt, rpt)],
                        out_hbm.at[cid, pl.ds(sid * rpt, rpt)])

        @pl.when(sid == 0)
        def _otail():
            pltpu.sync_copy(acc.at[pl.ds(_NS * rpt, otail)],
                            out_hbm.at[cid, pl.ds(_NS * rpt, otail)])

    return body(g, src2d, dst2d, zeros)


def _tc1(x, w1, deg_t):
    n, d = x.shape

    def body(x_ref, w_ref, deg_ref, g1_ref, dinv_ref):
        deg = deg_ref[:, 0:1] + deg_ref[:, 1:2] + 1.0
        dinv = lax.rsqrt(deg)
        h = jnp.dot(x_ref[...], w_ref[...], preferred_element_type=jnp.float32)
        g1_ref[...] = h * dinv
        dinv_ref[...] = dinv

    return pl.pallas_call(
        body,
        out_shape=(jax.ShapeDtypeStruct((n, d), jnp.float32),
                   jax.ShapeDtypeStruct((n, 1), jnp.float32)),
    )(x, w1, deg_t)


def _tc2(s1p, g1, dinv, b1, gnw, gnb, gna, w2):
    n, d = g1.shape

    def body(sp_ref, g_ref, di_ref, b_ref, w_ref, bt_ref, a_ref, w2_ref,
             x1_ref, g2_ref):
        s = sp_ref[0] + sp_ref[1] + g_ref[...]
        y = di_ref[...] * s + b_ref[...]
        mean = jnp.mean(y, axis=0, keepdims=True)
        o = y - a_ref[...] * mean
        var = jnp.mean(o * o, axis=0, keepdims=True)
        x1 = jnp.maximum(w_ref[...] * o * lax.rsqrt(var + _EPS) + bt_ref[...],
                         0.0)
        x1_ref[...] = x1
        g2_ref[...] = jnp.dot(x1, w2_ref[...],
                              preferred_element_type=jnp.float32) * di_ref[...]

    return pl.pallas_call(
        body,
        out_shape=(jax.ShapeDtypeStruct((n, d), jnp.float32),
                   jax.ShapeDtypeStruct((n, d), jnp.float32)),
    )(s1p, g1, dinv, b1, gnw, gnb, gna, w2)


def _tc3(s2p, g2, dinv, b2, gnw, gnb, gna, x1, wr, br):
    n, d = g2.shape

    def body(sp_ref, g_ref, di_ref, b_ref, w_ref, bt_ref, a_ref, x1_ref,
             wr_ref, br_ref, out_ref):
        s = sp_ref[0] + sp_ref[1] + g_ref[...]
        y = di_ref[...] * s + b_ref[...]
        mean = jnp.mean(y, axis=0, keepdims=True)
        o = y - a_ref[...] * mean
        var = jnp.mean(o * o, axis=0, keepdims=True)
        x2 = jnp.maximum(w_ref[...] * o * lax.rsqrt(var + _EPS) + bt_ref[...],
                         0.0)
        out_ref[...] = jnp.dot(x1_ref[...] + x2, wr_ref[...],
                               preferred_element_type=jnp.float32) + br_ref[...]

    return pl.pallas_call(
        body,
        out_shape=jax.ShapeDtypeStruct((n, d), jnp.float32),
    )(s2p, g2, dinv, b2, gnw, gnb, gna, x1, wr, br)


def kernel(x, edge_index, W1, b1, W2, b2, gn1_w, gn1_b, gn1_a, gn2_w, gn2_b,
           gn2_a, Wr, br):
    n, d = x.shape
    e = edge_index.shape[1]
    epad = _NW * _CPW * _K
    assert e <= epad and n % 16 == 0

    src = edge_index[0]
    dst = edge_index[1]
    pad = epad - e
    src2d = jnp.concatenate(
        [src, jnp.zeros((pad,), jnp.int32)]).reshape(epad // _K, _K)
    dst2d = jnp.concatenate(
        [dst, jnp.full((pad,), n, jnp.int32)]).reshape(epad // _K, _K)
    zeros = jnp.zeros((n + 16, d), jnp.float32)

    npad = -(-(n + 16) // 128) * 128
    degp = _deg_sc(dst2d, n, npad).reshape(_NC, npad)[:, :n]
    deg_t = degp.T

    g1, dinv = _tc1(x, W1, deg_t)
    s1p = _seg_sum_sc(g1, src2d, dst2d, zeros)
    x1, g2 = _tc2(s1p, g1, dinv, b1.reshape(1, d), gn1_w.reshape(1, d),
                  gn1_b.reshape(1, d), gn1_a.reshape(1, d), W2)
    s2p = _seg_sum_sc(g2, src2d, dst2d, zeros)
    return _tc3(s2p, g2, dinv, b2.reshape(1, d), gn2_w.reshape(1, d),
                gn2_b.reshape(1, d), gn2_a.reshape(1, d), x1, Wr, br.reshape(1, d))


# speedup vs baseline: 1.0044x; 1.0044x over previous
import functools

import jax
import jax.numpy as jnp
from jax import lax
from jax.experimental import pallas as pl
from jax.experimental.pallas import tpu as pltpu
from jax.experimental.pallas import tpu_sc as plsc

_EPS = 1e-5
_NC = 2
_NS = 16
_NW = _NC * _NS
_K = 128
_CPW = 80
_NBUF = 2


def _deg_sc(dst2d, n, npad):
    mesh = plsc.VectorSubcoreMesh(core_axis_name="c", subcore_axis_name="s")

    @functools.partial(
        pl.kernel,
        out_type=jax.ShapeDtypeStruct((_NC * npad,), jnp.float32),
        mesh=mesh,
        scratch_types=[
            pltpu.VMEM((_CPW, _K), jnp.int32),
            pltpu.VMEM((_K,), jnp.float32),
            pltpu.VMEM((npad,), jnp.float32),
            pltpu.VMEM_SHARED((npad,), jnp.float32),
            pltpu.SemaphoreType.DMA,
        ],
    )
    def body(dst_hbm, out_hbm, didx, ones, zbuf, acc, sem):
        cid = lax.axis_index("c")
        sid = lax.axis_index("s")
        wid = sid * _NC + cid
        pltpu.sync_copy(dst_hbm.at[pl.ds(wid * _CPW, _CPW)], didx)
        for j in range(_K // 16):
            ones[pl.ds(j * 16, 16)] = jnp.full((16,), 1.0, jnp.float32)

        @pl.when(sid == 0)
        def _zero():
            def zstep(i, c):
                zbuf[pl.ds(i * 16, 16)] = jnp.zeros((16,), jnp.float32)
                return c
            lax.fori_loop(0, npad // 16, zstep, 0)
            pltpu.sync_copy(zbuf, acc)

        plsc.subcore_barrier()

        def step(i, c):
            pltpu.async_copy(ones, acc.at[didx.at[i]], sem, add=True)
            return c

        lax.fori_loop(0, _CPW, step, 0)

        def drain(i, c):
            pltpu.make_async_copy(ones, acc.at[didx.at[0]], sem).wait()
            return c

        lax.fori_loop(0, _CPW, drain, 0)
        plsc.subcore_barrier()

        @pl.when(sid == 0)
        def _out():
            pltpu.sync_copy(acc, out_hbm.at[pl.ds(cid * npad, npad)])

    return body(dst2d)


def _seg_sum_sc(g, src2d, dst2d, zeros):
    n, d = g.shape
    nacc = zeros.shape[0]
    rpt = (n // _NS) // 8 * 8
    ztail = nacc - _NS * rpt
    otail = n - _NS * rpt
    cpq = 16
    nq = _CPW // cpq
    mesh = plsc.VectorSubcoreMesh(core_axis_name="c", subcore_axis_name="s")

    @functools.partial(
        pl.kernel,
        out_type=jax.ShapeDtypeStruct((_NC, n, d), jnp.float32),
        mesh=mesh,
        scratch_types=(
            [pltpu.VMEM((cpq, _K), jnp.int32),
             pltpu.VMEM((cpq, _K), jnp.int32),
             pltpu.VMEM_SHARED((nacc, d), jnp.float32)]
            + [pltpu.VMEM((_K, d), jnp.float32) for _ in range(_NBUF)]
            + [pltpu.SemaphoreType.DMA for _ in range(2 * _NBUF)]
        ),
    )
    def body(g_hbm, src_hbm, dst_hbm, z_hbm, out_hbm, sidx, didx, acc, *bufs):
        rows = bufs[:_NBUF]
        gsem = bufs[_NBUF:2 * _NBUF]
        ssem = bufs[2 * _NBUF:]
        cid = lax.axis_index("c")
        sid = lax.axis_index("s")
        wid = sid * _NC + cid

        pltpu.sync_copy(z_hbm.at[pl.ds(sid * rpt, rpt)],
                        acc.at[pl.ds(sid * rpt, rpt)])

        @pl.when(sid == 0)
        def _ztail():
            pltpu.sync_copy(z_hbm.at[pl.ds(_NS * rpt, ztail)],
                            acc.at[pl.ds(_NS * rpt, ztail)])

        plsc.subcore_barrier()

        nblk = cpq // _NBUF
        for q in range(nq):
            qbase = wid * _CPW + q * cpq
            pltpu.sync_copy(src_hbm.at[pl.ds(qbase, cpq)], sidx)
            pltpu.sync_copy(dst_hbm.at[pl.ds(qbase, cpq)], didx)
            for b in range(_NBUF):
                pltpu.async_copy(g_hbm.at[sidx.at[b]], rows[b], gsem[b])

            def blk(k, c):
                j0 = k * _NBUF
                for b in range(_NBUF):
                    j = j0 + b
                    pltpu.make_async_copy(g_hbm.at[sidx.at[j]], rows[b],
                                          gsem[b]).wait()
                    pltpu.async_copy(rows[b], acc.at[didx.at[j]], ssem[b],
                                     add=True)
                for b in range(_NBUF):
                    j = j0 + b

                    @pl.when(k < nblk - 1)
                    def _next():
                        pltpu.make_async_copy(rows[b], acc.at[didx.at[j]],
                                              ssem[b]).wait()
                        pltpu.async_copy(g_hbm.at[sidx.at[j + _NBUF]],
                                         rows[b], gsem[b])
                return c

            lax.fori_loop(0, nblk, blk, 0)
            for b in range(_NBUF):
                pltpu.make_async_copy(rows[b], acc.at[didx.at[0]],
                                      ssem[b]).wait()
        plsc.subcore_barrier()

        pltpu.sync_copy(acc.at[pl.ds(sid * rpt, rpt)],
                        out_hbm.at[cid, pl.ds(sid * rpt, rpt)])

        @pl.when(sid == 0)
        def _otail():
            pltpu.sync_copy(acc.at[pl.ds(_NS * rpt, otail)],
                            out_hbm.at[cid, pl.ds(_NS * rpt, otail)])

    return body(g, src2d, dst2d, zeros)


def _tc1(x, w1, deg_t):
    n, d = x.shape

    def body(x_ref, w_ref, deg_ref, g1_ref, dinv_ref):
        deg = deg_ref[:, 0:1] + deg_ref[:, 1:2] + 1.0
        dinv = lax.rsqrt(deg)
        h = jnp.dot(x_ref[...], w_ref[...], preferred_element_type=jnp.float32)
        g1_ref[...] = h * dinv
        dinv_ref[...] = dinv

    return pl.pallas_call(
        body,
        out_shape=(jax.ShapeDtypeStruct((n, d), jnp.float32),
                   jax.ShapeDtypeStruct((n, 1), jnp.float32)),
    )(x, w1, deg_t)


def _tc2(s1p, g1, dinv, b1, gnw, gnb, gna, w2):
    n, d = g1.shape

    def body(sp_ref, g_ref, di_ref, b_ref, w_ref, bt_ref, a_ref, w2_ref,
             x1_ref, g2_ref):
        s = sp_ref[0] + sp_ref[1] + g_ref[...]
        y = di_ref[...] * s + b_ref[...]
        mean = jnp.mean(y, axis=0, keepdims=True)
        o = y - a_ref[...] * mean
        var = jnp.mean(o * o, axis=0, keepdims=True)
        x1 = jnp.maximum(w_ref[...] * o * lax.rsqrt(var + _EPS) + bt_ref[...],
                         0.0)
        x1_ref[...] = x1
        g2_ref[...] = jnp.dot(x1, w2_ref[...],
                              preferred_element_type=jnp.float32) * di_ref[...]

    return pl.pallas_call(
        body,
        out_shape=(jax.ShapeDtypeStruct((n, d), jnp.float32),
                   jax.ShapeDtypeStruct((n, d), jnp.float32)),
    )(s1p, g1, dinv, b1, gnw, gnb, gna, w2)


def _tc3(s2p, g2, dinv, b2, gnw, gnb, gna, x1, wr, br):
    n, d = g2.shape

    def body(sp_ref, g_ref, di_ref, b_ref, w_ref, bt_ref, a_ref, x1_ref,
             wr_ref, br_ref, out_ref):
        s = sp_ref[0] + sp_ref[1] + g_ref[...]
        y = di_ref[...] * s + b_ref[...]
        mean = jnp.mean(y, axis=0, keepdims=True)
        o = y - a_ref[...] * mean
        var = jnp.mean(o * o, axis=0, keepdims=True)
        x2 = jnp.maximum(w_ref[...] * o * lax.rsqrt(var + _EPS) + bt_ref[...],
                         0.0)
        out_ref[...] = jnp.dot(x1_ref[...] + x2, wr_ref[...],
                               preferred_element_type=jnp.float32) + br_ref[...]

    return pl.pallas_call(
        body,
        out_shape=jax.ShapeDtypeStruct((n, d), jnp.float32),
    )(s2p, g2, dinv, b2, gnw, gnb, gna, x1, wr, br)


def kernel(x, edge_index, W1, b1, W2, b2, gn1_w, gn1_b, gn1_a, gn2_w, gn2_b,
           gn2_a, Wr, br):
    n, d = x.shape
    e = edge_index.shape[1]
    epad = _NW * _CPW * _K
    assert e <= epad and n % 16 == 0

    src = edge_index[0]
    dst = edge_index[1]
    pad = epad - e
    dummy = n + (jnp.arange(pad, dtype=jnp.int32) % 128)
    src2d = jnp.concatenate(
        [src, jnp.zeros((pad,), jnp.int32)]).reshape(epad // _K, _K)
    dst2d = jnp.concatenate([dst, dummy]).reshape(epad // _K, _K)
    zeros = jnp.zeros((n + 128, d), jnp.float32)

    npad = -(-(n + 128) // 128) * 128
    degp = _deg_sc(dst2d, n, npad).reshape(_NC, npad)[:, :n]
    deg_t = degp.T

    g1, dinv = _tc1(x, W1, deg_t)
    s1p = _seg_sum_sc(g1, src2d, dst2d, zeros)
    x1, g2 = _tc2(s1p, g1, dinv, b1.reshape(1, d), gn1_w.reshape(1, d),
                  gn1_b.reshape(1, d), gn1_a.reshape(1, d), W2)
    s2p = _seg_sum_sc(g2, src2d, dst2d, zeros)
    return _tc3(s2p, g2, dinv, b2.reshape(1, d), gn2_w.reshape(1, d),
                gn2_b.reshape(1, d), gn2_a.reshape(1, d), x1, Wr, br.reshape(1, d))


# speedup vs baseline: 1.0352x; 1.0307x over previous
import functools

import jax
import jax.numpy as jnp
from jax import lax
from jax.experimental import pallas as pl
from jax.experimental.pallas import tpu as pltpu
from jax.experimental.pallas import tpu_sc as plsc

_EPS = 1e-5
_NC = 2
_NS = 16
_NW = _NC * _NS
_K = 128
_CPW = 80
_NBUF = 2


def _deg_sc(dst2d, n, npad):
    mesh = plsc.VectorSubcoreMesh(core_axis_name="c", subcore_axis_name="s")

    @functools.partial(
        pl.kernel,
        out_type=jax.ShapeDtypeStruct((_NC * npad,), jnp.float32),
        mesh=mesh,
        scratch_types=[
            pltpu.VMEM((_CPW, _K), jnp.int32),
            pltpu.VMEM((_K,), jnp.float32),
            pltpu.VMEM((npad,), jnp.float32),
            pltpu.VMEM_SHARED((npad,), jnp.float32),
            pltpu.SemaphoreType.DMA,
        ],
    )
    def body(dst_hbm, out_hbm, didx, ones, zbuf, acc, sem):
        cid = lax.axis_index("c")
        sid = lax.axis_index("s")
        wid = sid * _NC + cid
        pltpu.sync_copy(dst_hbm.at[pl.ds(wid * _CPW, _CPW)], didx)
        for j in range(_K // 16):
            ones[pl.ds(j * 16, 16)] = jnp.full((16,), 1.0, jnp.float32)

        @pl.when(sid == 0)
        def _zero():
            def zstep(i, c):
                zbuf[pl.ds(i * 16, 16)] = jnp.zeros((16,), jnp.float32)
                return c
            lax.fori_loop(0, npad // 16, zstep, 0)
            pltpu.sync_copy(zbuf, acc)

        plsc.subcore_barrier()

        def step(i, c):
            pltpu.async_copy(ones, acc.at[didx.at[i]], sem, add=True)
            return c

        lax.fori_loop(0, _CPW, step, 0)

        def drain(i, c):
            pltpu.make_async_copy(ones, acc.at[didx.at[0]], sem).wait()
            return c

        lax.fori_loop(0, _CPW, drain, 0)
        plsc.subcore_barrier()

        @pl.when(sid == 0)
        def _out():
            pltpu.sync_copy(acc, out_hbm.at[pl.ds(cid * npad, npad)])

    return body(dst2d)


def _seg_sum_sc(g, src2d, dst2d, zeros):
    n, d = g.shape
    nacc = zeros.shape[0]
    rpt = (n // _NS) // 8 * 8
    ztail = nacc - _NS * rpt
    otail = n - _NS * rpt
    cpq = 16
    nq = _CPW // cpq
    mesh = plsc.VectorSubcoreMesh(core_axis_name="c", subcore_axis_name="s")

    @functools.partial(
        pl.kernel,
        out_type=jax.ShapeDtypeStruct((_NC, n, d), jnp.float32),
        mesh=mesh,
        scratch_types=(
            [pltpu.VMEM((cpq, _K), jnp.int32),
             pltpu.VMEM((cpq, _K), jnp.int32),
             pltpu.VMEM_SHARED((nacc, d), jnp.float32)]
            + [pltpu.VMEM((_K, d), jnp.float32) for _ in range(_NBUF)]
            + [pltpu.SemaphoreType.DMA for _ in range(2 * _NBUF)]
        ),
    )
    def body(g_hbm, src_hbm, dst_hbm, z_hbm, out_hbm, sidx, didx, acc, *bufs):
        rows = bufs[:_NBUF]
        gsem = bufs[_NBUF:2 * _NBUF]
        ssem = bufs[2 * _NBUF:]
        cid = lax.axis_index("c")
        sid = lax.axis_index("s")
        wid = sid * _NC + cid

        pltpu.sync_copy(z_hbm.at[pl.ds(sid * rpt, rpt)],
                        acc.at[pl.ds(sid * rpt, rpt)])

        @pl.when(sid == 0)
        def _ztail():
            pltpu.sync_copy(z_hbm.at[pl.ds(_NS * rpt, ztail)],
                            acc.at[pl.ds(_NS * rpt, ztail)])

        plsc.subcore_barrier()

        nblk = cpq // _NBUF
        for q in range(nq):
            qbase = wid * _CPW + q * cpq
            pltpu.sync_copy(src_hbm.at[pl.ds(qbase, cpq)], sidx)
            pltpu.sync_copy(dst_hbm.at[pl.ds(qbase, cpq)], didx)
            for b in range(_NBUF):
                pltpu.async_copy(g_hbm.at[sidx.at[b]], rows[b], gsem[b])

            def blk(k, c):
                j0 = k * _NBUF
                for b in range(_NBUF):
                    j = j0 + b
                    pltpu.make_async_copy(g_hbm.at[sidx.at[j]], rows[b],
                                          gsem[b]).wait()
                    pltpu.sync_copy(rows[b], acc.at[didx.at[j]], add=True)

                    @pl.when(k < nblk - 1)
                    def _next():
                        pltpu.async_copy(g_hbm.at[sidx.at[j + _NBUF]],
                                         rows[b], gsem[b])
                return c

            lax.fori_loop(0, nblk, blk, 0)
        plsc.subcore_barrier()

        pltpu.sync_copy(acc.at[pl.ds(sid * rpt, rpt)],
                        out_hbm.at[cid, pl.ds(sid * rpt, rpt)])

        @pl.when(sid == 0)
        def _otail():
            pltpu.sync_copy(acc.at[pl.ds(_NS * rpt, otail)],
                            out_hbm.at[cid, pl.ds(_NS * rpt, otail)])

    return body(g, src2d, dst2d, zeros)


def _tc1(x, w1, deg_t):
    n, d = x.shape

    def body(x_ref, w_ref, deg_ref, g1_ref, dinv_ref):
        deg = deg_ref[:, 0:1] + deg_ref[:, 1:2] + 1.0
        dinv = lax.rsqrt(deg)
        h = jnp.dot(x_ref[...], w_ref[...], preferred_element_type=jnp.float32)
        g1_ref[...] = h * dinv
        dinv_ref[...] = dinv

    return pl.pallas_call(
        body,
        out_shape=(jax.ShapeDtypeStruct((n, d), jnp.float32),
                   jax.ShapeDtypeStruct((n, 1), jnp.float32)),
    )(x, w1, deg_t)


def _tc2(s1p, g1, dinv, b1, gnw, gnb, gna, w2):
    n, d = g1.shape

    def body(sp_ref, g_ref, di_ref, b_ref, w_ref, bt_ref, a_ref, w2_ref,
             x1_ref, g2_ref):
        s = sp_ref[0] + sp_ref[1] + g_ref[...]
        y = di_ref[...] * s + b_ref[...]
        mean = jnp.mean(y, axis=0, keepdims=True)
        o = y - a_ref[...] * mean
        var = jnp.mean(o * o, axis=0, keepdims=True)
        x1 = jnp.maximum(w_ref[...] * o * lax.rsqrt(var + _EPS) + bt_ref[...],
                         0.0)
        x1_ref[...] = x1
        g2_ref[...] = jnp.dot(x1, w2_ref[...],
                              preferred_element_type=jnp.float32) * di_ref[...]

    return pl.pallas_call(
        body,
        out_shape=(jax.ShapeDtypeStruct((n, d), jnp.float32),
                   jax.ShapeDtypeStruct((n, d), jnp.float32)),
    )(s1p, g1, dinv, b1, gnw, gnb, gna, w2)


def _tc3(s2p, g2, dinv, b2, gnw, gnb, gna, x1, wr, br):
    n, d = g2.shape

    def body(sp_ref, g_ref, di_ref, b_ref, w_ref, bt_ref, a_ref, x1_ref,
             wr_ref, br_ref, out_ref):
        s = sp_ref[0] + sp_ref[1] + g_ref[...]
        y = di_ref[...] * s + b_ref[...]
        mean = jnp.mean(y, axis=0, keepdims=True)
        o = y - a_ref[...] * mean
        var = jnp.mean(o * o, axis=0, keepdims=True)
        x2 = jnp.maximum(w_ref[...] * o * lax.rsqrt(var + _EPS) + bt_ref[...],
                         0.0)
        out_ref[...] = jnp.dot(x1_ref[...] + x2, wr_ref[...],
                               preferred_element_type=jnp.float32) + br_ref[...]

    return pl.pallas_call(
        body,
        out_shape=jax.ShapeDtypeStruct((n, d), jnp.float32),
    )(s2p, g2, dinv, b2, gnw, gnb, gna, x1, wr, br)


def kernel(x, edge_index, W1, b1, W2, b2, gn1_w, gn1_b, gn1_a, gn2_w, gn2_b,
           gn2_a, Wr, br):
    n, d = x.shape
    e = edge_index.shape[1]
    epad = _NW * _CPW * _K
    assert e <= epad and n % 16 == 0

    src = edge_index[0]
    dst = edge_index[1]
    pad = epad - e
    dummy = n + (jnp.arange(pad, dtype=jnp.int32) % 128)
    src2d = jnp.concatenate(
        [src, jnp.zeros((pad,), jnp.int32)]).reshape(epad // _K, _K)
    dst2d = jnp.concatenate([dst, dummy]).reshape(epad // _K, _K)
    zeros = jnp.zeros((n + 128, d), jnp.float32)

    npad = -(-(n + 128) // 128) * 128
    degp = _deg_sc(dst2d, n, npad).reshape(_NC, npad)[:, :n]
    deg_t = degp.T

    g1, dinv = _tc1(x, W1, deg_t)
    s1p = _seg_sum_sc(g1, src2d, dst2d, zeros)
    x1, g2 = _tc2(s1p, g1, dinv, b1.reshape(1, d), gn1_w.reshape(1, d),
                  gn1_b.reshape(1, d), gn1_a.reshape(1, d), W2)
    s2p = _seg_sum_sc(g2, src2d, dst2d, zeros)
    return _tc3(s2p, g2, dinv, b2.reshape(1, d), gn2_w.reshape(1, d),
                gn2_b.reshape(1, d), gn2_a.reshape(1, d), x1, Wr, br.reshape(1, d))


# speedup vs baseline: 1.0615x; 1.0254x over previous
import functools

import jax
import jax.numpy as jnp
from jax import lax
from jax.experimental import pallas as pl
from jax.experimental.pallas import tpu as pltpu
from jax.experimental.pallas import tpu_sc as plsc

_EPS = 1e-5
_NC = 2
_NS = 16
_NW = _NC * _NS
_K = 128
_CPW = 80
_NBUF = 2


def _deg_sc(dst2d, n, npad):
    mesh = plsc.VectorSubcoreMesh(core_axis_name="c", subcore_axis_name="s")

    @functools.partial(
        pl.kernel,
        out_type=jax.ShapeDtypeStruct((_NC * npad,), jnp.float32),
        mesh=mesh,
        scratch_types=[
            pltpu.VMEM((_CPW, _K), jnp.int32),
            pltpu.VMEM((_K,), jnp.float32),
            pltpu.VMEM((npad,), jnp.float32),
            pltpu.VMEM_SHARED((npad,), jnp.float32),
            pltpu.SemaphoreType.DMA,
        ],
    )
    def body(dst_hbm, out_hbm, didx, ones, zbuf, acc, sem):
        cid = lax.axis_index("c")
        sid = lax.axis_index("s")
        wid = sid * _NC + cid
        pltpu.sync_copy(dst_hbm.at[pl.ds(wid * _CPW, _CPW)], didx)
        for j in range(_K // 16):
            ones[pl.ds(j * 16, 16)] = jnp.full((16,), 1.0, jnp.float32)

        @pl.when(sid == 0)
        def _zero():
            def zstep(i, c):
                zbuf[pl.ds(i * 16, 16)] = jnp.zeros((16,), jnp.float32)
                return c
            lax.fori_loop(0, npad // 16, zstep, 0)
            pltpu.sync_copy(zbuf, acc)

        plsc.subcore_barrier()

        def step(i, c):
            pltpu.async_copy(ones, acc.at[didx.at[i]], sem, add=True)
            return c

        lax.fori_loop(0, _CPW, step, 0)

        def drain(i, c):
            pltpu.make_async_copy(ones, acc.at[didx.at[0]], sem).wait()
            return c

        lax.fori_loop(0, _CPW, drain, 0)
        plsc.subcore_barrier()

        @pl.when(sid == 0)
        def _out():
            pltpu.sync_copy(acc, out_hbm.at[pl.ds(cid * npad, npad)])

    return body(dst2d)


def _seg_sum_sc(g, src2d, dst2d, zeros):
    n, d = g.shape
    nacc = zeros.shape[0]
    rpt = (n // _NS) // 8 * 8
    ztail = nacc - _NS * rpt
    otail = n - _NS * rpt
    epw = _CPW * _K
    mesh = plsc.VectorSubcoreMesh(core_axis_name="c", subcore_axis_name="s")

    @functools.partial(
        pl.kernel,
        out_type=jax.ShapeDtypeStruct((_NC, n, d), jnp.float32),
        mesh=mesh,
        scratch_types=(
            [pltpu.VMEM_SHARED((nacc, d), jnp.float32)]
            + [pltpu.VMEM((_K,), jnp.int32) for _ in range(2 * _NBUF)]
            + [pltpu.VMEM((_K, d), jnp.float32) for _ in range(_NBUF)]
            + [pltpu.SemaphoreType.DMA for _ in range(_NBUF)]
        ),
    )
    def body(g_hbm, src_hbm, dst_hbm, z_hbm, out_hbm, acc, *bufs):
        sidx = bufs[:_NBUF]
        didx = bufs[_NBUF:2 * _NBUF]
        rows = bufs[2 * _NBUF:3 * _NBUF]
        gsem = bufs[3 * _NBUF:]
        cid = lax.axis_index("c")
        sid = lax.axis_index("s")
        wid = sid * _NC + cid
        base = wid * epw

        for b in range(_NBUF):
            pltpu.sync_copy(src_hbm.at[pl.ds(base + b * _K, _K)], sidx[b])
            pltpu.sync_copy(dst_hbm.at[pl.ds(base + b * _K, _K)], didx[b])
            pltpu.async_copy(g_hbm.at[sidx[b]], rows[b], gsem[b])
        pltpu.sync_copy(z_hbm.at[pl.ds(sid * rpt, rpt)],
                        acc.at[pl.ds(sid * rpt, rpt)])

        @pl.when(sid == 0)
        def _ztail():
            pltpu.sync_copy(z_hbm.at[pl.ds(_NS * rpt, ztail)],
                            acc.at[pl.ds(_NS * rpt, ztail)])

        plsc.subcore_barrier()

        nblk = _CPW // _NBUF

        def blk(k, c):
            j0 = k * _NBUF
            for b in range(_NBUF):
                j = j0 + b
                pltpu.make_async_copy(g_hbm.at[sidx[b]], rows[b],
                                      gsem[b]).wait()
                pltpu.sync_copy(rows[b], acc.at[didx[b]], add=True)

                @pl.when(j + _NBUF < _CPW)
                def _next():
                    nxt = base + (j + _NBUF) * _K
                    pltpu.sync_copy(src_hbm.at[pl.ds(nxt, _K)], sidx[b])
                    pltpu.sync_copy(dst_hbm.at[pl.ds(nxt, _K)], didx[b])
                    pltpu.async_copy(g_hbm.at[sidx[b]], rows[b], gsem[b])
            return c

        lax.fori_loop(0, nblk, blk, 0)
        plsc.subcore_barrier()

        pltpu.sync_copy(acc.at[pl.ds(sid * rpt, rpt)],
                        out_hbm.at[cid, pl.ds(sid * rpt, rpt)])

        @pl.when(sid == 0)
        def _otail():
            pltpu.sync_copy(acc.at[pl.ds(_NS * rpt, otail)],
                            out_hbm.at[cid, pl.ds(_NS * rpt, otail)])

    return body(g, src2d, dst2d, zeros)


def _tc1(x, w1, deg_t):
    n, d = x.shape

    def body(x_ref, w_ref, deg_ref, g1_ref, dinv_ref):
        deg = deg_ref[:, 0:1] + deg_ref[:, 1:2] + 1.0
        dinv = lax.rsqrt(deg)
        h = jnp.dot(x_ref[...], w_ref[...], preferred_element_type=jnp.float32)
        g1_ref[...] = h * dinv
        dinv_ref[...] = dinv

    return pl.pallas_call(
        body,
        out_shape=(jax.ShapeDtypeStruct((n, d), jnp.float32),
                   jax.ShapeDtypeStruct((n, 1), jnp.float32)),
    )(x, w1, deg_t)


def _tc2(s1p, g1, dinv, b1, gnw, gnb, gna, w2):
    n, d = g1.shape

    def body(sp_ref, g_ref, di_ref, b_ref, w_ref, bt_ref, a_ref, w2_ref,
             x1_ref, g2_ref):
        s = sp_ref[0] + sp_ref[1] + g_ref[...]
        y = di_ref[...] * s + b_ref[...]
        mean = jnp.mean(y, axis=0, keepdims=True)
        o = y - a_ref[...] * mean
        var = jnp.mean(o * o, axis=0, keepdims=True)
        x1 = jnp.maximum(w_ref[...] * o * lax.rsqrt(var + _EPS) + bt_ref[...],
                         0.0)
        x1_ref[...] = x1
        g2_ref[...] = jnp.dot(x1, w2_ref[...],
                              preferred_element_type=jnp.float32) * di_ref[...]

    return pl.pallas_call(
        body,
        out_shape=(jax.ShapeDtypeStruct((n, d), jnp.float32),
                   jax.ShapeDtypeStruct((n, d), jnp.float32)),
    )(s1p, g1, dinv, b1, gnw, gnb, gna, w2)


def _tc3(s2p, g2, dinv, b2, gnw, gnb, gna, x1, wr, br):
    n, d = g2.shape

    def body(sp_ref, g_ref, di_ref, b_ref, w_ref, bt_ref, a_ref, x1_ref,
             wr_ref, br_ref, out_ref):
        s = sp_ref[0] + sp_ref[1] + g_ref[...]
        y = di_ref[...] * s + b_ref[...]
        mean = jnp.mean(y, axis=0, keepdims=True)
        o = y - a_ref[...] * mean
        var = jnp.mean(o * o, axis=0, keepdims=True)
        x2 = jnp.maximum(w_ref[...] * o * lax.rsqrt(var + _EPS) + bt_ref[...],
                         0.0)
        out_ref[...] = jnp.dot(x1_ref[...] + x2, wr_ref[...],
                               preferred_element_type=jnp.float32) + br_ref[...]

    return pl.pallas_call(
        body,
        out_shape=jax.ShapeDtypeStruct((n, d), jnp.float32),
    )(s2p, g2, dinv, b2, gnw, gnb, gna, x1, wr, br)


def kernel(x, edge_index, W1, b1, W2, b2, gn1_w, gn1_b, gn1_a, gn2_w, gn2_b,
           gn2_a, Wr, br):
    n, d = x.shape
    e = edge_index.shape[1]
    epad = _NW * _CPW * _K
    assert e <= epad and n % 16 == 0

    src = edge_index[0]
    dst = edge_index[1]
    pad = epad - e
    dummy = n + (jnp.arange(pad, dtype=jnp.int32) % 128)
    srcp = jnp.concatenate([src, jnp.zeros((pad,), jnp.int32)])
    dstp = jnp.concatenate([dst, dummy])
    dst2d = dstp.reshape(epad // _K, _K)
    zeros = jnp.zeros((n + 128, d), jnp.float32)

    npad = -(-(n + 128) // 128) * 128
    degp = _deg_sc(dst2d, n, npad).reshape(_NC, npad)[:, :n]
    deg_t = degp.T

    g1, dinv = _tc1(x, W1, deg_t)
    s1p = _seg_sum_sc(g1, srcp, dstp, zeros)
    x1, g2 = _tc2(s1p, g1, dinv, b1.reshape(1, d), gn1_w.reshape(1, d),
                  gn1_b.reshape(1, d), gn1_a.reshape(1, d), W2)
    s2p = _seg_sum_sc(g2, srcp, dstp, zeros)
    return _tc3(s2p, g2, dinv, b2.reshape(1, d), gn2_w.reshape(1, d),
                gn2_b.reshape(1, d), gn2_a.reshape(1, d), x1, Wr, br.reshape(1, d))
